# baseline (speedup 1.0000x reference)
import functools

import jax
import jax.numpy as jnp
from jax import lax
from jax.experimental import pallas as pl
from jax.experimental.pallas import tpu as pltpu
from jax.experimental.pallas import tpu_sc as plsc

B = 8
N = 10000
E = 160000
F = 2
T = 12
C = 16

NTILES = 16
NP = 10240
NPT = NP // NTILES
EPT = E // NTILES
CH = 80
NCH = EPT // CH
KH = (B // 2) * F * T


def _rsqrt_newton(x):
    xi = lax.bitcast_convert_type(x, jnp.int32)
    yi = jnp.int32(0x5F3759DF) - (xi >> 1)
    y = lax.bitcast_convert_type(yi, jnp.float32)
    for _ in range(3):
        y = y * (1.5 - 0.5 * x * y * y)
    return y


def _splat(v16, u):
    return jnp.take_along_axis(v16, jnp.full((16,), u, jnp.int32), axis=0)


def _sc_body(ei2_hbm, ew_hbm, xst_hbm,
             y_hbm,
             deg_sh, dinv_sh, y_sh,
             dinv_v, degv, dinvv, snv,
             row2d_all, col2d_all, ew_all,
             gbufa, gbufb, sema, semb, semsa, semsb):
    c = lax.axis_index("c")
    s = lax.axis_index("s")
    nbase = pl.multiple_of(s * NPT, 8)
    ebase = pl.multiple_of(s * EPT, 8)
    z16 = jnp.zeros((16,), jnp.float32)

    pltpu.sync_copy(ei2_hbm.at[0, pl.ds(s * NCH, NCH), :], row2d_all)
    pltpu.sync_copy(ei2_hbm.at[1, pl.ds(s * NCH, NCH), :], col2d_all)
    pltpu.sync_copy(ew_hbm.at[pl.ds(ebase, EPT)], ew_all)
    for j in range(NPT // 16):
        degv[pl.ds(16 * j, 16)] = z16
    pltpu.sync_copy(degv, deg_sh.at[pl.ds(nbase, NPT)])
    plsc.subcore_barrier()

    def deg_chunk(j, carry):
        pltpu.async_copy(ew_all.at[pl.ds(j * CH, CH)],
                         deg_sh.at[col2d_all.at[j]], sema, add=True)
        return carry

    lax.fori_loop(0, NCH, deg_chunk, 0)

    def deg_drain(j, carry):
        pltpu.make_async_copy(ew_all.at[pl.ds(j * CH, CH)],
                              deg_sh.at[col2d_all.at[j]], sema).wait()
        return carry

    lax.fori_loop(0, NCH, deg_drain, 0)
    plsc.subcore_barrier()

    pltpu.sync_copy(deg_sh.at[pl.ds(nbase, NPT)], degv)
    for j in range(NPT // 16):
        xv = degv[pl.ds(16 * j, 16)] + 1.0
        y = _rsqrt_newton(xv)
        dinvv[pl.ds(16 * j, 16)] = y
        snv[pl.ds(16 * j, 16)] = y * y
    pltpu.sync_copy(dinvv, dinv_sh.at[pl.ds(nbase, NPT)])

    coff = c * N
    for m in range(NPT // CH):
        pltpu.sync_copy(xst_hbm.at[pl.ds(coff + nbase + m * CH, CH), :], gbufa)

        def sl_scale(g, carry, m=m):
            s16 = snv[pl.ds(m * CH + 16 * g, 16)]
            for u in range(16):
                spl = _splat(s16, u)
                e = 16 * g + u
                for jj in range(KH // 16):
                    gbufa[e, pl.ds(16 * jj, 16)] = (
                        gbufa[e, pl.ds(16 * jj, 16)] * spl)
            return carry

        lax.fori_loop(0, CH // 16, sl_scale, 0)
        pltpu.sync_copy(gbufa, y_sh.at[pl.ds(nbase + m * CH, CH), :])
    plsc.subcore_barrier()

    pltpu.sync_copy(dinv_sh, dinv_v)

    def nrm_chunk(j, carry):
        for g in range(CH // 16):
            o = j * CH + 16 * g
            r16 = row2d_all[j, pl.ds(16 * g, 16)]
            c16 = col2d_all[j, pl.ds(16 * g, 16)]
            dr = plsc.load_gather(dinv_v, [r16])
            dc = plsc.load_gather(dinv_v, [c16])
            ew_all[pl.ds(o, 16)] = dr * ew_all[pl.ds(o, 16)] * dc
            row2d_all[j, pl.ds(16 * g, 16)] = r16 + coff
        return carry

    lax.fori_loop(0, NCH, nrm_chunk, 0)

    def gather_start(chunk, buf, sem):
        pltpu.async_copy(xst_hbm.at[row2d_all.at[chunk]], buf, sem)

    def gather_wait(chunk, buf, sem):
        pltpu.make_async_copy(xst_hbm.at[row2d_all.at[chunk]], buf, sem).wait()

    def scale_rows(buf, eoff):
        def scale16(g, carry2):
            nrm16 = ew_all[pl.ds(eoff + 16 * g, 16)]
            for u in range(16):
                spl = _splat(nrm16, u)
                e = 16 * g + u
                for jj in range(KH // 16):
                    buf[e, pl.ds(16 * jj, 16)] = buf[e, pl.ds(16 * jj, 16)] * spl
            return carry2

        lax.fori_loop(0, CH // 16, scale16, 0)

    def scatter_start(chunk, buf, sem):
        pltpu.async_copy(buf, y_sh.at[col2d_all.at[chunk]], sem, add=True)

    def scatter_wait(chunk, buf, sem):
        pltpu.make_async_copy(buf, y_sh.at[col2d_all.at[chunk]], sem).wait()

    gather_start(0, gbufa, sema)
    gather_start(1, gbufb, semb)

    def pair_body(p, carry):
        a = 2 * p
        gather_wait(a, gbufa, sema)
        scale_rows(gbufa, a * CH)
        scatter_start(a, gbufa, semsa)
        gather_wait(a + 1, gbufb, semb)
        scale_rows(gbufb, (a + 1) * CH)
        scatter_start(a + 1, gbufb, semsb)
        scatter_wait(a, gbufa, semsa)
        gather_start(a + 2, gbufa, sema)

        @pl.when(p < (NCH - 1) // 2 - 1)
        def _():
            scatter_wait(a + 1, gbufb, semsb)
            gather_start(a + 3, gbufb, semb)

        return carry

    lax.fori_loop(0, (NCH - 1) // 2, pair_body, 0)
    scatter_wait(NCH - 2, gbufb, semsb)
    gather_wait(NCH - 1, gbufa, sema)
    scale_rows(gbufa, (NCH - 1) * CH)
    scatter_start(NCH - 1, gbufa, semsa)
    scatter_wait(NCH - 1, gbufa, semsa)
    plsc.subcore_barrier()

    obase = pl.multiple_of(coff + nbase, 8)

    @pl.when(s < NTILES - 1)
    def _():
        pltpu.sync_copy(y_sh.at[pl.ds(nbase, NPT), :], y_hbm.at[pl.ds(obase, NPT), :])

    @pl.when(s == NTILES - 1)
    def _():
        tail = N - (NTILES - 1) * NPT
        pltpu.sync_copy(y_sh.at[pl.ds(nbase, tail), :], y_hbm.at[pl.ds(obase, tail), :])


_sc_kernel = functools.partial(
    pl.kernel,
    mesh=plsc.VectorSubcoreMesh(core_axis_name="c", subcore_axis_name="s"),
    compiler_params=pltpu.CompilerParams(needs_layout_passes=False,
                                         use_tc_tiling_on_sc=False),
    out_type=[
        jax.ShapeDtypeStruct((2 * N, KH), jnp.float32),
    ],
    scratch_types=[
        pltpu.VMEM_SHARED((NP,), jnp.float32),
        pltpu.VMEM_SHARED((NP,), jnp.float32),
        pltpu.VMEM_SHARED((NP, KH), jnp.float32),
        pltpu.VMEM((NP,), jnp.float32),
        pltpu.VMEM((NPT,), jnp.float32),
        pltpu.VMEM((NPT,), jnp.float32),
        pltpu.VMEM((NPT,), jnp.float32),
        pltpu.VMEM((NCH, CH), jnp.int32),
        pltpu.VMEM((NCH, CH), jnp.int32),
        pltpu.VMEM((EPT,), jnp.float32),
        pltpu.VMEM((CH, KH), jnp.float32),
        pltpu.VMEM((CH, KH), jnp.float32),
        pltpu.SemaphoreType.DMA,
        pltpu.SemaphoreType.DMA,
        pltpu.SemaphoreType.DMA,
        pltpu.SemaphoreType.DMA,
    ],
)(_sc_body)


NB = 1000


def _tc_body(ya_ref, yb_ref, gz_ref, gh_ref, czt_ref,
             cht_ref, p_ref, wlin_ref, blin_ref, out_ref):
    gz = gz_ref[...]
    gh = gh_ref[...]
    czt = czt_ref[...]
    cht = cht_ref[...]
    pmat = p_ref[...]
    wlin = wlin_ref[...]
    blin = blin_ref[...]
    for h in range(2):
        yf = (ya_ref, yb_ref)[h][...]
        for bb in range(4):
            yc = yf[:, bb * 24:(bb + 1) * 24]
            uz = jnp.dot(yc, gz, preferred_element_type=jnp.float32) + czt
            uh = jnp.dot(yc, gh, preferred_element_type=jnp.float32) + cht
            ht = (1.0 - jax.nn.sigmoid(uz)) * jnp.tanh(uh)
            hacc = jnp.dot(ht, pmat, preferred_element_type=jnp.float32)
            o = jnp.dot(jnp.maximum(hacc, 0.0), wlin,
                        preferred_element_type=jnp.float32) + blin
            out_ref[4 * h + bb, :, :] = o


def kernel(x, edge_index, edge_weight, attention, Wz, bz, Lz, lbz, Wr, br, Lr,
           lbr, Wh, bh, Lh, lbh, Wlin, blin):
    ei2 = edge_index.reshape(2, E // CH, CH)
    xst = jnp.transpose(x.reshape(2, 4, N, F * T), (0, 2, 1, 3)).reshape(2 * N, KH)

    (y2,) = _sc_kernel(ei2, edge_weight, xst)

    mz = Wz @ Lz[:C]
    cz = bz @ Lz[:C] + lbz
    mh = Wh @ Lh[:C]
    ch = bh @ Lh[:C] + lbh
    probs = jax.nn.softmax(attention)
    eyeT = jnp.eye(T, dtype=jnp.float32)
    eyeC = jnp.eye(C, dtype=jnp.float32)
    gz = jnp.einsum('fc,tu->ftuc', mz, eyeT).reshape(F * T, T * C)
    gh = jnp.einsum('fc,tu->ftuc', mh, eyeT).reshape(F * T, T * C)
    czt = jnp.tile(cz, T).reshape(1, T * C)
    cht = jnp.tile(ch, T).reshape(1, T * C)
    pmat = jnp.einsum('t,cu->tcu', probs, eyeC).reshape(T * C, C)
    blin2 = blin.reshape(1, T)

    grid = (N // NB,)
    out = pl.pallas_call(
        _tc_body,
        grid=grid,
        in_specs=[
            pl.BlockSpec((NB, KH), lambda i: (i, 0)),
            pl.BlockSpec((NB, KH), lambda i: (N // NB + i, 0)),
            pl.BlockSpec((F * T, T * C), lambda i: (0, 0)),
            pl.BlockSpec((F * T, T * C), lambda i: (0, 0)),
            pl.BlockSpec((1, T * C), lambda i: (0, 0)),
            pl.BlockSpec((1, T * C), lambda i: (0, 0)),
            pl.BlockSpec((T * C, C), lambda i: (0, 0)),
            pl.BlockSpec((C, T), lambda i: (0, 0)),
            pl.BlockSpec((1, T), lambda i: (0, 0)),
        ],
        out_specs=pl.BlockSpec((B, NB, T), lambda i: (0, i, 0)),
        out_shape=jax.ShapeDtypeStruct((B, N, T), jnp.float32),
    )(y2, y2, gz, gh, czt, cht, pmat, Wlin, blin2)
    return out


# speedup vs baseline: 1.0016x; 1.0016x over previous
import functools

import jax
import jax.numpy as jnp
from jax import lax
from jax.experimental import pallas as pl
from jax.experimental.pallas import tpu as pltpu
from jax.experimental.pallas import tpu_sc as plsc

B = 8
N = 10000
E = 160000
F = 2
T = 12
C = 16

NTILES = 16
NP = 10240
NPT = NP // NTILES
EPT = E // NTILES
CH = 80
NCH = EPT // CH
KH = (B // 2) * F * T


def _rsqrt_newton(x):
    xi = lax.bitcast_convert_type(x, jnp.int32)
    yi = jnp.int32(0x5F3759DF) - (xi >> 1)
    y = lax.bitcast_convert_type(yi, jnp.float32)
    for _ in range(3):
        y = y * (1.5 - 0.5 * x * y * y)
    return y


def _splat(v16, u):
    return jnp.take_along_axis(v16, jnp.full((16,), u, jnp.int32), axis=0)


def _sc_body(ei_hbm, ew_hbm, xst_hbm,
             y_hbm,
             deg_sh, dinv_sh, y_sh,
             dinv_v, degv, dinvv, snv,
             row_all, col_all, ew_all,
             gbufa, gbufb, sema, semb, semsa, semsb):
    c = lax.axis_index("c")
    s = lax.axis_index("s")
    nbase = pl.multiple_of(s * NPT, 8)
    ebase = pl.multiple_of(s * EPT, 8)
    z16 = jnp.zeros((16,), jnp.float32)

    pltpu.sync_copy(ei_hbm.at[0, pl.ds(ebase, EPT)], row_all)
    pltpu.sync_copy(ei_hbm.at[1, pl.ds(ebase, EPT)], col_all)
    pltpu.sync_copy(ew_hbm.at[pl.ds(ebase, EPT)], ew_all)
    for j in range(NPT // 16):
        degv[pl.ds(16 * j, 16)] = z16
    pltpu.sync_copy(degv, deg_sh.at[pl.ds(nbase, NPT)])
    plsc.subcore_barrier()

    def deg_chunk(j, carry):
        pltpu.async_copy(ew_all.at[pl.ds(j * CH, CH)],
                         deg_sh.at[col_all.at[pl.ds(j * CH, CH)]], sema, add=True)
        return carry

    lax.fori_loop(0, NCH, deg_chunk, 0)

    def deg_drain(j, carry):
        pltpu.make_async_copy(ew_all.at[pl.ds(j * CH, CH)],
                              deg_sh.at[col_all.at[pl.ds(j * CH, CH)]], sema).wait()
        return carry

    lax.fori_loop(0, NCH, deg_drain, 0)
    plsc.subcore_barrier()

    pltpu.sync_copy(deg_sh.at[pl.ds(nbase, NPT)], degv)
    for j in range(NPT // 16):
        xv = degv[pl.ds(16 * j, 16)] + 1.0
        y = _rsqrt_newton(xv)
        dinvv[pl.ds(16 * j, 16)] = y
        snv[pl.ds(16 * j, 16)] = y * y
    pltpu.sync_copy(dinvv, dinv_sh.at[pl.ds(nbase, NPT)])

    coff = c * N
    for m in range(NPT // CH):
        pltpu.sync_copy(xst_hbm.at[pl.ds(coff + nbase + m * CH, CH), :], gbufa)

        def sl_scale(g, carry, m=m):
            s16 = snv[pl.ds(m * CH + 16 * g, 16)]
            for u in range(16):
                spl = _splat(s16, u)
                e = 16 * g + u
                for jj in range(KH // 16):
                    gbufa[e, pl.ds(16 * jj, 16)] = (
                        gbufa[e, pl.ds(16 * jj, 16)] * spl)
            return carry

        lax.fori_loop(0, CH // 16, sl_scale, 0)
        pltpu.sync_copy(gbufa, y_sh.at[pl.ds(nbase + m * CH, CH), :])
    plsc.subcore_barrier()

    pltpu.sync_copy(dinv_sh, dinv_v)

    def nrm_chunk(j, carry):
        for g in range(CH // 16):
            o = j * CH + 16 * g
            r16 = row_all[pl.ds(o, 16)]
            c16 = col_all[pl.ds(o, 16)]
            dr = plsc.load_gather(dinv_v, [r16])
            dc = plsc.load_gather(dinv_v, [c16])
            ew_all[pl.ds(o, 16)] = dr * ew_all[pl.ds(o, 16)] * dc
            row_all[pl.ds(o, 16)] = r16 + coff
        return carry

    lax.fori_loop(0, NCH, nrm_chunk, 0)

    def gather_start(chunk, buf, sem):
        pltpu.async_copy(xst_hbm.at[row_all.at[pl.ds(chunk * CH, CH)]], buf, sem)

    def gather_wait(chunk, buf, sem):
        pltpu.make_async_copy(xst_hbm.at[row_all.at[pl.ds(chunk * CH, CH)]], buf, sem).wait()

    def scale_rows(buf, eoff):
        def scale16(g, carry2):
            nrm16 = ew_all[pl.ds(eoff + 16 * g, 16)]
            for u in range(16):
                spl = _splat(nrm16, u)
                e = 16 * g + u
                for jj in range(KH // 16):
                    buf[e, pl.ds(16 * jj, 16)] = buf[e, pl.ds(16 * jj, 16)] * spl
            return carry2

        lax.fori_loop(0, CH // 16, scale16, 0)

    def scatter_start(chunk, buf, sem):
        pltpu.async_copy(buf, y_sh.at[col_all.at[pl.ds(chunk * CH, CH)]], sem, add=True)

    def scatter_wait(chunk, buf, sem):
        pltpu.make_async_copy(buf, y_sh.at[col_all.at[pl.ds(chunk * CH, CH)]], sem).wait()

    gather_start(0, gbufa, sema)
    gather_start(1, gbufb, semb)

    def pair_body(p, carry):
        a = 2 * p
        gather_wait(a, gbufa, sema)
        scale_rows(gbufa, a * CH)
        scatter_start(a, gbufa, semsa)
        gather_wait(a + 1, gbufb, semb)
        scale_rows(gbufb, (a + 1) * CH)
        scatter_start(a + 1, gbufb, semsb)
        scatter_wait(a, gbufa, semsa)
        gather_start(a + 2, gbufa, sema)

        @pl.when(p < (NCH - 1) // 2 - 1)
        def _():
            scatter_wait(a + 1, gbufb, semsb)
            gather_start(a + 3, gbufb, semb)

        return carry

    lax.fori_loop(0, (NCH - 1) // 2, pair_body, 0)
    scatter_wait(NCH - 2, gbufb, semsb)
    gather_wait(NCH - 1, gbufa, sema)
    scale_rows(gbufa, (NCH - 1) * CH)
    scatter_start(NCH - 1, gbufa, semsa)
    scatter_wait(NCH - 1, gbufa, semsa)
    plsc.subcore_barrier()

    obase = pl.multiple_of(coff + nbase, 8)

    @pl.when(s < NTILES - 1)
    def _():
        pltpu.sync_copy(y_sh.at[pl.ds(nbase, NPT), :], y_hbm.at[pl.ds(obase, NPT), :])

    @pl.when(s == NTILES - 1)
    def _():
        tail = N - (NTILES - 1) * NPT
        pltpu.sync_copy(y_sh.at[pl.ds(nbase, tail), :], y_hbm.at[pl.ds(obase, tail), :])


_sc_kernel = functools.partial(
    pl.kernel,
    mesh=plsc.VectorSubcoreMesh(core_axis_name="c", subcore_axis_name="s"),
    compiler_params=pltpu.CompilerParams(needs_layout_passes=False,
                                         use_tc_tiling_on_sc=False),
    out_type=[
        jax.ShapeDtypeStruct((2 * N, KH), jnp.float32),
    ],
    scratch_types=[
        pltpu.VMEM_SHARED((NP,), jnp.float32),
        pltpu.VMEM_SHARED((NP,), jnp.float32),
        pltpu.VMEM_SHARED((NP, KH), jnp.float32),
        pltpu.VMEM((NP,), jnp.float32),
        pltpu.VMEM((NPT,), jnp.float32),
        pltpu.VMEM((NPT,), jnp.float32),
        pltpu.VMEM((NPT,), jnp.float32),
        pltpu.VMEM((EPT,), jnp.int32),
        pltpu.VMEM((EPT,), jnp.int32),
        pltpu.VMEM((EPT,), jnp.float32),
        pltpu.VMEM((CH, KH), jnp.float32),
        pltpu.VMEM((CH, KH), jnp.float32),
        pltpu.SemaphoreType.DMA,
        pltpu.SemaphoreType.DMA,
        pltpu.SemaphoreType.DMA,
        pltpu.SemaphoreType.DMA,
    ],
)(_sc_body)


NB = 1000


def _tc_body(ya_ref, yb_ref, gz_ref, gh_ref, czt_ref,
             cht_ref, p_ref, wlin_ref, blin_ref, out_ref):
    gz = gz_ref[...]
    gh = gh_ref[...]
    czt = czt_ref[...]
    cht = cht_ref[...]
    pmat = p_ref[...]
    wlin = wlin_ref[...]
    blin = blin_ref[...]
    for h in range(2):
        yf = (ya_ref, yb_ref)[h][...]
        for bb in range(4):
            yc = yf[:, bb * 24:(bb + 1) * 24]
            uz = jnp.dot(yc, gz, preferred_element_type=jnp.float32) + czt
            uh = jnp.dot(yc, gh, preferred_element_type=jnp.float32) + cht
            ht = (1.0 - jax.nn.sigmoid(uz)) * jnp.tanh(uh)
            hacc = jnp.dot(ht, pmat, preferred_element_type=jnp.float32)
            o = jnp.dot(jnp.maximum(hacc, 0.0), wlin,
                        preferred_element_type=jnp.float32) + blin
            out_ref[4 * h + bb, :, :] = o


def kernel(x, edge_index, edge_weight, attention, Wz, bz, Lz, lbz, Wr, br, Lr,
           lbr, Wh, bh, Lh, lbh, Wlin, blin):
    xst = jnp.transpose(x.reshape(2, 4, N, F * T), (0, 2, 1, 3)).reshape(2 * N, KH)

    (y2,) = _sc_kernel(edge_index, edge_weight, xst)

    mz = Wz @ Lz[:C]
    cz = bz @ Lz[:C] + lbz
    mh = Wh @ Lh[:C]
    ch = bh @ Lh[:C] + lbh
    probs = jax.nn.softmax(attention)
    eyeT = jnp.eye(T, dtype=jnp.float32)
    eyeC = jnp.eye(C, dtype=jnp.float32)
    gz = jnp.einsum('fc,tu->ftuc', mz, eyeT).reshape(F * T, T * C)
    gh = jnp.einsum('fc,tu->ftuc', mh, eyeT).reshape(F * T, T * C)
    czt = jnp.tile(cz, T).reshape(1, T * C)
    cht = jnp.tile(ch, T).reshape(1, T * C)
    pmat = jnp.einsum('t,cu->tcu', probs, eyeC).reshape(T * C, C)
    blin2 = blin.reshape(1, T)

    grid = (N // NB,)
    out = pl.pallas_call(
        _tc_body,
        grid=grid,
        in_specs=[
            pl.BlockSpec((NB, KH), lambda i: (i, 0)),
            pl.BlockSpec((NB, KH), lambda i: (N // NB + i, 0)),
            pl.BlockSpec((F * T, T * C), lambda i: (0, 0)),
            pl.BlockSpec((F * T, T * C), lambda i: (0, 0)),
            pl.BlockSpec((1, T * C), lambda i: (0, 0)),
            pl.BlockSpec((1, T * C), lambda i: (0, 0)),
            pl.BlockSpec((T * C, C), lambda i: (0, 0)),
            pl.BlockSpec((C, T), lambda i: (0, 0)),
            pl.BlockSpec((1, T), lambda i: (0, 0)),
        ],
        out_specs=pl.BlockSpec((B, NB, T), lambda i: (0, i, 0)),
        out_shape=jax.ShapeDtypeStruct((B, N, T), jnp.float32),
    )(y2, y2, gz, gh, czt, cht, pmat, Wlin, blin2)
    return out


# speedup vs baseline: 1.0956x; 1.0938x over previous
import functools

import jax
import jax.numpy as jnp
from jax import lax
from jax.experimental import pallas as pl
from jax.experimental.pallas import tpu as pltpu
from jax.experimental.pallas import tpu_sc as plsc

B = 8
N = 10000
E = 160000
F = 2
T = 12
C = 16

NTILES = 16
NP = 10240
NPT = NP // NTILES
EPT = E // NTILES
CH = 80
NCH = EPT // CH
KH = (B // 2) * F * T


def _rsqrt_newton(x):
    xi = lax.bitcast_convert_type(x, jnp.int32)
    yi = jnp.int32(0x5F3759DF) - (xi >> 1)
    y = lax.bitcast_convert_type(yi, jnp.float32)
    for _ in range(3):
        y = y * (1.5 - 0.5 * x * y * y)
    return y


def _sc_body(row_hbm, col2d_hbm, ew_hbm, xst_hbm,
             y_hbm, sn_hbm,
             deg_sh, dinv_sh, y_sh,
             dinv_v, degv, dinvv, snv,
             row_all, col2d_all, ew_all,
             gbufa, gbufb, sema, semb, semsa, semsb):
    c = lax.axis_index("c")
    s = lax.axis_index("s")
    nbase = pl.multiple_of(s * NPT, 8)
    ebase = pl.multiple_of(s * EPT, 8)

    pltpu.sync_copy(row_hbm.at[pl.ds(ebase, EPT)], row_all)
    pltpu.sync_copy(col2d_hbm.at[pl.ds(s * NCH, NCH), :], col2d_all)
    pltpu.sync_copy(ew_hbm.at[pl.ds(ebase, EPT)], ew_all)
    z16 = jnp.zeros((16,), jnp.float32)

    def zrow(r, carry):
        for jj in range(KH // 16):
            gbufa[r, pl.ds(16 * jj, 16)] = z16
        return carry

    lax.fori_loop(0, CH, zrow, 0)
    for j in range(NPT // 16):
        degv[pl.ds(16 * j, 16)] = z16
    pltpu.sync_copy(degv, deg_sh.at[pl.ds(nbase, NPT)])
    for m in range(NPT // CH):
        pltpu.sync_copy(gbufa, y_sh.at[pl.ds(nbase + m * CH, CH), :])
    plsc.subcore_barrier()

    def deg_chunk(j, carry):
        pltpu.async_copy(ew_all.at[pl.ds(j * CH, CH)],
                         deg_sh.at[col2d_all.at[j]], sema, add=True)
        return carry

    lax.fori_loop(0, NCH, deg_chunk, 0)

    def deg_drain(j, carry):
        pltpu.make_async_copy(ew_all.at[pl.ds(j * CH, CH)],
                              deg_sh.at[col2d_all.at[j]], sema).wait()
        return carry

    lax.fori_loop(0, NCH, deg_drain, 0)
    plsc.subcore_barrier()

    pltpu.sync_copy(deg_sh.at[pl.ds(nbase, NPT)], degv)
    for j in range(NPT // 16):
        xv = degv[pl.ds(16 * j, 16)] + 1.0
        y = _rsqrt_newton(xv)
        dinvv[pl.ds(16 * j, 16)] = y
        snv[pl.ds(16 * j, 16)] = y * y
    pltpu.sync_copy(dinvv, dinv_sh.at[pl.ds(nbase, NPT)])

    @pl.when(jnp.logical_and(c == 0, s < NTILES - 1))
    def _():
        pltpu.sync_copy(snv, sn_hbm.at[pl.ds(nbase, NPT)])

    @pl.when(jnp.logical_and(c == 0, s == NTILES - 1))
    def _():
        tail = N - (NTILES - 1) * NPT
        pltpu.sync_copy(snv.at[pl.ds(0, tail)], sn_hbm.at[pl.ds(nbase, tail)])

    plsc.subcore_barrier()

    pltpu.sync_copy(dinv_sh, dinv_v)

    coff = c * N

    def nrm_chunk(j, carry):
        for g in range(CH // 16):
            o = j * CH + 16 * g
            r16 = row_all[pl.ds(o, 16)]
            c16 = col2d_all[j, pl.ds(16 * g, 16)]
            dr = plsc.load_gather(dinv_v, [r16])
            dc = plsc.load_gather(dinv_v, [c16])
            ew_all[pl.ds(o, 16)] = dr * ew_all[pl.ds(o, 16)] * dc
            row_all[pl.ds(o, 16)] = r16 + coff
        return carry

    lax.fori_loop(0, NCH, nrm_chunk, 0)

    def gather_start(chunk, buf, sem):
        pltpu.async_copy(xst_hbm.at[row_all.at[pl.ds(chunk * CH, CH)]],
                         buf, sem)

    def gather_wait(chunk, buf, sem):
        pltpu.make_async_copy(xst_hbm.at[row_all.at[pl.ds(chunk * CH, CH)]],
                              buf, sem).wait()

    def scale_rows(buf, eoff):
        def scale16(g, carry2):
            nrm16 = ew_all[pl.ds(eoff + 16 * g, 16)]
            for u in range(16):
                spl = jnp.take_along_axis(
                    nrm16, jnp.full((16,), u, jnp.int32), axis=0)
                e = 16 * g + u
                for jj in range(KH // 16):
                    buf[e, pl.ds(16 * jj, 16)] = buf[e, pl.ds(16 * jj, 16)] * spl
            return carry2

        lax.fori_loop(0, CH // 16, scale16, 0)

    def scatter_start(chunk, buf, sem):
        pltpu.async_copy(buf, y_sh.at[col2d_all.at[chunk]], sem, add=True)

    def scatter_wait(chunk, buf, sem):
        pltpu.make_async_copy(buf, y_sh.at[col2d_all.at[chunk]], sem).wait()

    gather_start(0, gbufa, sema)
    gather_start(1, gbufb, semb)

    def pair_body(p, carry):
        a = 2 * p
        gather_wait(a, gbufa, sema)
        scale_rows(gbufa, a * CH)
        scatter_start(a, gbufa, semsa)
        gather_wait(a + 1, gbufb, semb)
        scale_rows(gbufb, (a + 1) * CH)
        scatter_start(a + 1, gbufb, semsb)
        scatter_wait(a, gbufa, semsa)
        gather_start(a + 2, gbufa, sema)

        @pl.when(p < (NCH - 1) // 2 - 1)
        def _():
            scatter_wait(a + 1, gbufb, semsb)
            gather_start(a + 3, gbufb, semb)

        return carry

    lax.fori_loop(0, (NCH - 1) // 2, pair_body, 0)
    scatter_wait(NCH - 2, gbufb, semsb)
    gather_wait(NCH - 1, gbufa, sema)
    scale_rows(gbufa, (NCH - 1) * CH)
    scatter_start(NCH - 1, gbufa, semsa)
    scatter_wait(NCH - 1, gbufa, semsa)
    plsc.subcore_barrier()

    @pl.when(s < NTILES - 1)
    def _():
        pltpu.sync_copy(y_sh.at[pl.ds(nbase, NPT), :], y_hbm.at[c, pl.ds(nbase, NPT), :])

    @pl.when(s == NTILES - 1)
    def _():
        tail = N - (NTILES - 1) * NPT
        pltpu.sync_copy(y_sh.at[pl.ds(nbase, tail), :], y_hbm.at[c, pl.ds(nbase, tail), :])


_sc_kernel = functools.partial(
    pl.kernel,
    mesh=plsc.VectorSubcoreMesh(core_axis_name="c", subcore_axis_name="s"),
    compiler_params=pltpu.CompilerParams(needs_layout_passes=False,
                                         use_tc_tiling_on_sc=False),
    out_type=[
        jax.ShapeDtypeStruct((2, N, KH), jnp.float32),
        jax.ShapeDtypeStruct((N,), jnp.float32),
    ],
    scratch_types=[
        pltpu.VMEM_SHARED((NP,), jnp.float32),
        pltpu.VMEM_SHARED((NP,), jnp.float32),
        pltpu.VMEM_SHARED((NP, KH), jnp.float32),
        pltpu.VMEM((NP,), jnp.float32),
        pltpu.VMEM((NPT,), jnp.float32),
        pltpu.VMEM((NPT,), jnp.float32),
        pltpu.VMEM((NPT,), jnp.float32),
        pltpu.VMEM((EPT,), jnp.int32),
        pltpu.VMEM((NCH, CH), jnp.int32),
        pltpu.VMEM((EPT,), jnp.float32),
        pltpu.VMEM((CH, KH), jnp.float32),
        pltpu.VMEM((CH, KH), jnp.float32),
        pltpu.SemaphoreType.DMA,
        pltpu.SemaphoreType.DMA,
        pltpu.SemaphoreType.DMA,
        pltpu.SemaphoreType.DMA,
    ],
)(_sc_body)


NB = 1000


def _tc_body(ya_ref, yb_ref, xa_ref, xb_ref, sn_ref, gz_ref, gh_ref, czt_ref,
             cht_ref, p_ref, wlin_ref, blin_ref, out_ref):
    sn = sn_ref[...]
    gz = gz_ref[...]
    gh = gh_ref[...]
    czt = czt_ref[...]
    cht = cht_ref[...]
    pmat = p_ref[...]
    wlin = wlin_ref[...]
    blin = blin_ref[...]
    for h in range(2):
        y_ref, x_ref = ((ya_ref, xa_ref), (yb_ref, xb_ref))[h]
        yf = y_ref[...] + sn * x_ref[...]
        for bb in range(4):
            yc = yf[:, bb * 24:(bb + 1) * 24]
            uz = jnp.dot(yc, gz, preferred_element_type=jnp.float32) + czt
            uh = jnp.dot(yc, gh, preferred_element_type=jnp.float32) + cht
            ht = (1.0 - jax.nn.sigmoid(uz)) * jnp.tanh(uh)
            hacc = jnp.dot(ht, pmat, preferred_element_type=jnp.float32)
            o = jnp.dot(jnp.maximum(hacc, 0.0), wlin,
                        preferred_element_type=jnp.float32) + blin
            out_ref[4 * h + bb, :, :] = o


def kernel(x, edge_index, edge_weight, attention, Wz, bz, Lz, lbz, Wr, br, Lr,
           lbr, Wh, bh, Lh, lbh, Wlin, blin):
    row = edge_index[0]
    colf = edge_index[1]
    col2d = colf.reshape(E // CH, CH)
    xst = jnp.transpose(x.reshape(2, 4, N, F * T), (0, 2, 1, 3)).reshape(2 * N, KH)

    y_agg, sn = _sc_kernel(row, col2d, edge_weight, xst)
    y2 = y_agg.reshape(2 * N, KH)

    mz = Wz @ Lz[:C]
    cz = bz @ Lz[:C] + lbz
    mh = Wh @ Lh[:C]
    ch = bh @ Lh[:C] + lbh
    probs = jax.nn.softmax(attention)
    eyeT = jnp.eye(T, dtype=jnp.float32)
    eyeC = jnp.eye(C, dtype=jnp.float32)
    gz = jnp.einsum('fc,tu->ftuc', mz, eyeT).reshape(F * T, T * C)
    gh = jnp.einsum('fc,tu->ftuc', mh, eyeT).reshape(F * T, T * C)
    czt = jnp.tile(cz, T).reshape(1, T * C)
    cht = jnp.tile(ch, T).reshape(1, T * C)
    pmat = jnp.einsum('t,cu->tcu', probs, eyeC).reshape(T * C, C)
    sn2 = sn.reshape(N, 1)
    blin2 = blin.reshape(1, T)

    grid = (N // NB,)
    out = pl.pallas_call(
        _tc_body,
        grid=grid,
        in_specs=[
            pl.BlockSpec((NB, KH), lambda i: (i, 0)),
            pl.BlockSpec((NB, KH), lambda i: (N // NB + i, 0)),
            pl.BlockSpec((NB, KH), lambda i: (i, 0)),
            pl.BlockSpec((NB, KH), lambda i: (N // NB + i, 0)),
            pl.BlockSpec((NB, 1), lambda i: (i, 0)),
            pl.BlockSpec((F * T, T * C), lambda i: (0, 0)),
            pl.BlockSpec((F * T, T * C), lambda i: (0, 0)),
            pl.BlockSpec((1, T * C), lambda i: (0, 0)),
            pl.BlockSpec((1, T * C), lambda i: (0, 0)),
            pl.BlockSpec((T * C, C), lambda i: (0, 0)),
            pl.BlockSpec((C, T), lambda i: (0, 0)),
            pl.BlockSpec((1, T), lambda i: (0, 0)),
        ],
        out_specs=pl.BlockSpec((B, NB, T), lambda i: (0, i, 0)),
        out_shape=jax.ShapeDtypeStruct((B, N, T), jnp.float32),
    )(y2, y2, xst, xst, sn2, gz, gh, czt, cht, pmat, Wlin, blin2)
    return out
